# baseline (speedup 1.0000x reference)
import jax
import jax.numpy as jnp
from jax.experimental import pallas as pl

_NUM_CODES = 8192
_CODE_DIM = 64
_BETA = 0.25
_TOKEN_TILE = 512
_CHUNK = 4096


def _vq_tile_kernel(z_ref, z2_ref, zz_ref, cbb_ref, cc_ref,
                    zq_ref, idx_ref, sse_ref):
    i = pl.program_id(0)
    z = z_ref[...]
    z2 = z2_ref[...]
    zz = zz_ref[0, 0, :][:, None]
    cbb = cbb_ref[...]
    cc = cc_ref[0, 0, :]

    mm = jax.lax.dot_general(
        z2, cbb, (((1,), (1,)), ((), ())),
        preferred_element_type=jnp.float32)
    dist = (zz + cc[None, :]) - mm

    best_idx = jnp.zeros((_TOKEN_TILE,), dtype=jnp.int32)
    carry = jnp.full((_TOKEN_TILE,), jnp.inf, dtype=jnp.float32)
    iota = jax.lax.broadcasted_iota(jnp.int32, (_TOKEN_TILE, _CHUNK), 1)
    for q in range(_NUM_CODES // _CHUNK):
        sl = dist[:, q * _CHUNK:(q + 1) * _CHUNK]
        m = jnp.min(sl, axis=1)
        a = jnp.min(jnp.where(sl == m[:, None], iota, _NUM_CODES),
                    axis=1).astype(jnp.int32) + q * _CHUNK
        upd = m < carry
        best_idx = jnp.where(upd, a, best_idx)
        carry = jnp.where(
            upd, m.astype(jnp.bfloat16).astype(jnp.float32), carry)

    onehot = (jax.lax.broadcasted_iota(jnp.int32, (_TOKEN_TILE, _NUM_CODES), 1)
              == best_idx[:, None]).astype(jnp.bfloat16)
    zq = jax.lax.dot_general(
        onehot, cbb, (((1,), (0,)), ((), ())),
        preferred_element_type=jnp.float32)
    diff = zq - z
    zq_ref[...] = z + diff
    idx_ref[0, 0, :] = best_idx
    s = jnp.sum(diff * diff).reshape(1, 1)

    @pl.when(i == 0)
    def _init():
        sse_ref[...] = s

    @pl.when(i > 0)
    def _acc():
        sse_ref[...] = sse_ref[...] + s


def kernel(z_e, codebook):
    b, h, w, d = z_e.shape
    n = b * h * w
    z_flat = z_e.reshape(n, d)
    nb = n // _TOKEN_TILE

    zz = jnp.sum(z_flat ** 2, axis=1, keepdims=True)
    cc = jnp.sum(codebook ** 2, axis=1)
    z2 = (2.0 * z_flat).astype(jnp.bfloat16)
    cbb = codebook.astype(jnp.bfloat16)

    zq_st, idx3, sse = pl.pallas_call(
        _vq_tile_kernel,
        grid=(nb,),
        in_specs=[
            pl.BlockSpec((_TOKEN_TILE, d), lambda i: (i, 0)),
            pl.BlockSpec((_TOKEN_TILE, d), lambda i: (i, 0)),
            pl.BlockSpec((1, 1, _TOKEN_TILE), lambda i: (i, 0, 0)),
            pl.BlockSpec((_NUM_CODES, d), lambda i: (0, 0)),
            pl.BlockSpec((1, 1, _NUM_CODES), lambda i: (0, 0, 0)),
        ],
        out_specs=[
            pl.BlockSpec((_TOKEN_TILE, d), lambda i: (i, 0)),
            pl.BlockSpec((1, 1, _TOKEN_TILE), lambda i: (i, 0, 0)),
            pl.BlockSpec((1, 1), lambda i: (0, 0)),
        ],
        out_shape=[
            jax.ShapeDtypeStruct((n, d), jnp.float32),
            jax.ShapeDtypeStruct((nb, 1, _TOKEN_TILE), jnp.int32),
            jax.ShapeDtypeStruct((1, 1), jnp.float32),
        ],
    )(z_flat, z2, zz.reshape(nb, 1, _TOKEN_TILE), cbb,
      cc.reshape(1, 1, _NUM_CODES))

    mse = sse[0, 0] / jnp.float32(n * d)
    vq_loss = mse + _BETA * mse
    return zq_st.reshape(z_e.shape), vq_loss, idx3.reshape(b, h, w)


# speedup vs baseline: 1.3758x; 1.3758x over previous
import functools

import jax
import jax.numpy as jnp
from jax import lax
from jax.experimental import pallas as pl
from jax.experimental.pallas import tpu as pltpu, tpu_sc as plsc

_NUM_CODES = 8192
_CODE_DIM = 64
_DP = 128
_BETA = 0.25
_TOKEN_TILE = 256
_CHUNK = 4096
_N = 16384

_NC, _NS = 2, 16
_NW = _NC * _NS
_B_PER_W = _N // _NW


def _select_kernel(z2_ref, zz_ref, cbb_ref, cc_ref, idx_ref):
    z2 = z2_ref[...]
    zz = zz_ref[0, 0, :][:, None]
    cbb = cbb_ref[...]
    cc = cc_ref[0, 0, :]

    mm = jax.lax.dot_general(
        z2, cbb, (((1,), (1,)), ((), ())),
        preferred_element_type=jnp.float32)
    dist = (zz + cc[None, :]) - mm

    best_idx = jnp.zeros((_TOKEN_TILE,), dtype=jnp.int32)
    carry = jnp.full((_TOKEN_TILE,), jnp.inf, dtype=jnp.float32)
    iota = jax.lax.broadcasted_iota(jnp.int32, (_TOKEN_TILE, _CHUNK), 1)
    for q in range(_NUM_CODES // _CHUNK):
        sl = dist[:, q * _CHUNK:(q + 1) * _CHUNK]
        m = jnp.min(sl, axis=1)
        a = jnp.min(jnp.where(sl == m[:, None], iota, _NUM_CODES),
                    axis=1).astype(jnp.int32) + q * _CHUNK
        upd = m < carry
        best_idx = jnp.where(upd, a, best_idx)
        carry = jnp.where(
            upd, m.astype(jnp.bfloat16).astype(jnp.float32), carry)

    idx_ref[0, 0, :] = best_idx


_sc_mesh = plsc.VectorSubcoreMesh(core_axis_name="c", subcore_axis_name="s")


@functools.partial(
    pl.kernel, mesh=_sc_mesh,
    out_type=jax.ShapeDtypeStruct((_N, _DP), jnp.float32),
    scratch_types=[
        pltpu.VMEM((_B_PER_W,), jnp.int32),
        pltpu.VMEM((_B_PER_W, _DP), jnp.float32),
        pltpu.SemaphoreType.DMA,
    ],
)
def _sc_gather(table_hbm, idx_hbm, out_hbm, idx_v, rows_v, sem):
    wid = lax.axis_index("s") * _NC + lax.axis_index("c")
    base = wid * _B_PER_W
    pltpu.sync_copy(idx_hbm.at[pl.ds(base, _B_PER_W)], idx_v)
    pltpu.async_copy(table_hbm.at[idx_v], rows_v, sem).wait()
    pltpu.sync_copy(rows_v, out_hbm.at[pl.ds(base, _B_PER_W)])


def _st_loss_kernel(z_ref, zqp_ref, zq_ref, sse_ref):
    i = pl.program_id(0)
    z = z_ref[...]
    zq = zqp_ref[:, :_CODE_DIM]
    diff = zq - z
    zq_ref[...] = z + diff
    s = jnp.sum(diff * diff).reshape(1, 1)

    @pl.when(i == 0)
    def _init():
        sse_ref[...] = s

    @pl.when(i > 0)
    def _acc():
        sse_ref[...] = sse_ref[...] + s


def kernel(z_e, codebook):
    b, h, w, d = z_e.shape
    n = b * h * w
    z_flat = z_e.reshape(n, d)
    nb = n // _TOKEN_TILE

    zz = jnp.sum(z_flat ** 2, axis=1, keepdims=True)
    cc = jnp.sum(codebook ** 2, axis=1)
    z2 = (2.0 * z_flat).astype(jnp.bfloat16)
    cbb = codebook.astype(jnp.bfloat16)
    cbp = jnp.pad(codebook, ((0, 0), (0, _DP - d)))

    idx3 = pl.pallas_call(
        _select_kernel,
        grid=(nb,),
        in_specs=[
            pl.BlockSpec((_TOKEN_TILE, d), lambda i: (i, 0)),
            pl.BlockSpec((1, 1, _TOKEN_TILE), lambda i: (i, 0, 0)),
            pl.BlockSpec((_NUM_CODES, d), lambda i: (0, 0)),
            pl.BlockSpec((1, 1, _NUM_CODES), lambda i: (0, 0, 0)),
        ],
        out_specs=pl.BlockSpec((1, 1, _TOKEN_TILE), lambda i: (i, 0, 0)),
        out_shape=jax.ShapeDtypeStruct((nb, 1, _TOKEN_TILE), jnp.int32),
    )(z2, zz.reshape(nb, 1, _TOKEN_TILE), cbb, cc.reshape(1, 1, _NUM_CODES))

    zqp = _sc_gather(cbp, idx3.reshape(n))

    zq_st, sse = pl.pallas_call(
        _st_loss_kernel,
        grid=(nb,),
        in_specs=[
            pl.BlockSpec((_TOKEN_TILE, d), lambda i: (i, 0)),
            pl.BlockSpec((_TOKEN_TILE, _DP), lambda i: (i, 0)),
        ],
        out_specs=[
            pl.BlockSpec((_TOKEN_TILE, d), lambda i: (i, 0)),
            pl.BlockSpec((1, 1), lambda i: (0, 0)),
        ],
        out_shape=[
            jax.ShapeDtypeStruct((n, d), jnp.float32),
            jax.ShapeDtypeStruct((1, 1), jnp.float32),
        ],
    )(z_flat, zqp)

    mse = sse[0, 0] / jnp.float32(n * d)
    vq_loss = mse + _BETA * mse
    return zq_st.reshape(z_e.shape), vq_loss, idx3.reshape(b, h, w)
